# Initial kernel scaffold; baseline (speedup 1.0000x reference)
import jax
import jax.numpy as jnp
from jax.experimental import pallas as pl


def kernel(frames):
    raise NotImplementedError("write your pallas kernel here")


# speedup vs baseline: 1.2643x; 1.2643x over previous
import jax
import jax.numpy as jnp
from jax.experimental import pallas as pl
from jax.experimental.pallas import tpu as pltpu

ALPHA = 4


def _gather_copy(idx_ref, in_ref, out_ref):
    del idx_ref
    out_ref[...] = in_ref[...]


def kernel(frames):
    C, T, H, W = frames.shape
    n_slow = T // ALPHA
    idx = jnp.linspace(0.0, float(T - 1), n_slow).astype(jnp.int32)

    slow = pl.pallas_call(
        _gather_copy,
        grid_spec=pltpu.PrefetchScalarGridSpec(
            num_scalar_prefetch=1,
            grid=(C, n_slow),
            in_specs=[
                pl.BlockSpec((1, 1, H, W), lambda c, j, idx_ref: (c, idx_ref[j], 0, 0)),
            ],
            out_specs=pl.BlockSpec((1, 1, H, W), lambda c, j, idx_ref: (c, j, 0, 0)),
        ),
        out_shape=jax.ShapeDtypeStruct((C, n_slow, H, W), frames.dtype),
    )(idx, frames)

    return (slow, frames)
